# baseline (speedup 1.0000x reference)
import functools

import jax
import jax.numpy as jnp
from jax import lax
from jax.experimental import pallas as pl
from jax.experimental.pallas import tpu as pltpu
from jax.experimental.pallas import tpu_sc as plsc

DIM = 768
N_ART = 32 * 512
N_Q = 32 * 64
N_OPT = 32 * 5 * 48

NUM_CORES = 2
NUM_SUBCORES = 16
NW = NUM_CORES * NUM_SUBCORES


def _sc_gather(word_emb, ids, row_off, rows_per_w, chunk):
    n_rows = rows_per_w * NW
    n = rows_per_w // chunk
    mesh = plsc.VectorSubcoreMesh(core_axis_name="c", subcore_axis_name="s")

    @functools.partial(
        pl.kernel,
        mesh=mesh,
        out_type=jax.ShapeDtypeStruct((n_rows, DIM), jnp.float32),
        scratch_types=[
            pltpu.VMEM((rows_per_w,), jnp.int32),
            pltpu.VMEM((chunk, DIM), jnp.float32),
            pltpu.VMEM((chunk, DIM), jnp.float32),
            pltpu.SemaphoreType.DMA,
            pltpu.SemaphoreType.DMA,
            pltpu.SemaphoreType.DMA,
            pltpu.SemaphoreType.DMA,
        ],
    )
    def k(table_hbm, idx_hbm, out_hbm, idx_v, buf0, buf1, g0, g1, w0, w1):
        wid = lax.axis_index("s") * NUM_CORES + lax.axis_index("c")
        base = wid * rows_per_w
        pltpu.sync_copy(idx_hbm.at[pl.ds(row_off + base, rows_per_w)], idx_v)

        bufs = (buf0, buf1)
        gsems = (g0, g1)
        wsems = (w0, w1)

        def gather(c):
            cp = pltpu.make_async_copy(
                table_hbm.at[idx_v.at[pl.ds(c * chunk, chunk)]],
                bufs[c % 2], gsems[c % 2])
            cp.start()
            return cp

        def write(c):
            cp = pltpu.make_async_copy(
                bufs[c % 2],
                out_hbm.at[pl.ds(base + c * chunk, chunk)],
                wsems[c % 2])
            cp.start()
            return cp

        gathers = [gather(0)]
        writes = []
        for c in range(n):
            if c + 1 < n:
                if c >= 1:
                    writes[c - 1].wait()
                gathers.append(gather(c + 1))
            gathers[c].wait()
            writes.append(write(c))
        writes[n - 1].wait()
        if n > 1:
            writes[n - 2].wait()

    return k(word_emb, ids)


def _ln_body(g_ref, add_ref, gam_ref, bet_ref, o_ref):
    x = g_ref[...] + add_ref[...]
    mu = jnp.mean(x, axis=1, keepdims=True)
    xc = x - mu
    var = jnp.mean(xc * xc, axis=1, keepdims=True)
    o_ref[...] = xc * lax.rsqrt(var + 1e-12) * gam_ref[...] + bet_ref[...]


def _ln_alias_body(g_ref, _old_ref, add_ref, gam_ref, bet_ref, o_ref):
    _ln_body(g_ref, add_ref, gam_ref, bet_ref, o_ref)


def _ln_call(gathered, addtab, gamma2d, beta2d, block, out_rows=None):
    nrows = gathered.shape[0]
    grid = nrows // block
    if out_rows is None:
        out_rows = nrows
    return pl.pallas_call(
        _ln_body,
        grid=(grid,),
        in_specs=[
            pl.BlockSpec((block, DIM), lambda i: (i, 0)),
            pl.BlockSpec((block, DIM), lambda i: (0, 0)),
            pl.BlockSpec((1, DIM), lambda i: (0, 0)),
            pl.BlockSpec((1, DIM), lambda i: (0, 0)),
        ],
        out_specs=pl.BlockSpec((block, DIM), lambda i: (i, 0)),
        out_shape=jax.ShapeDtypeStruct((out_rows, DIM), jnp.float32),
    )(gathered, addtab, gamma2d, beta2d)


def _ln_call_alias(gathered, partial_out, addtab, gamma2d, beta2d, block,
                   blk_off):
    nrows = gathered.shape[0]
    grid = nrows // block
    return pl.pallas_call(
        _ln_alias_body,
        grid=(grid,),
        in_specs=[
            pl.BlockSpec((block, DIM), lambda i: (i, 0)),
            pl.BlockSpec((block, DIM), lambda i: (i + blk_off, 0)),
            pl.BlockSpec((block, DIM), lambda i: (0, 0)),
            pl.BlockSpec((1, DIM), lambda i: (0, 0)),
            pl.BlockSpec((1, DIM), lambda i: (0, 0)),
        ],
        out_specs=pl.BlockSpec((block, DIM), lambda i: (i + blk_off, 0)),
        out_shape=jax.ShapeDtypeStruct(partial_out.shape, jnp.float32),
        input_output_aliases={1: 0},
    )(gathered, partial_out, addtab, gamma2d, beta2d)


def _after(x, dep):
    return lax.optimization_barrier((x, dep))[0]


def kernel(article_tokens, question_tokens, options_tokens, word_emb,
           pos_emb, tok_type_emb, gamma, beta):
    art_ids = article_tokens.reshape(-1).astype(jnp.int32)
    q_ids = question_tokens.reshape(-1).astype(jnp.int32)
    opt_ids = options_tokens.reshape(-1).astype(jnp.int32)
    half = N_ART // 2

    ga1 = _sc_gather(word_emb, art_ids, 0, rows_per_w=256, chunk=64)
    go = _sc_gather(_after(word_emb, ga1), opt_ids, 0, rows_per_w=240, chunk=40)
    ga2 = _sc_gather(_after(word_emb, go), art_ids, half, rows_per_w=256,
                     chunk=64)
    gq = _sc_gather(_after(word_emb, ga2), q_ids, 0, rows_per_w=64, chunk=64)

    addvec = pos_emb + tok_type_emb[0]
    a_add = jnp.tile(addvec, (2, 1))
    q_add = jnp.tile(addvec[:64], (16, 1))
    o_add = jnp.tile(addvec[:1], (1280, 1))
    g2 = gamma.reshape(1, DIM)
    b2 = beta.reshape(1, DIM)

    art1 = _ln_call(ga1, a_add, g2, b2, block=1024, out_rows=N_ART)
    opt = _ln_call(go, o_add, g2, b2, block=1280)
    art = _ln_call_alias(ga2, art1, a_add, g2, b2, block=1024,
                         blk_off=half // 1024)
    q = _ln_call(gq, q_add, g2, b2, block=1024)

    return (art.reshape(32, 512, DIM),
            q.reshape(32, 64, DIM),
            opt.reshape(32, 5, 48, DIM))


# speedup vs baseline: 1.1317x; 1.1317x over previous
import functools

import jax
import jax.numpy as jnp
from jax import lax
from jax.experimental import pallas as pl
from jax.experimental.pallas import tpu as pltpu
from jax.experimental.pallas import tpu_sc as plsc

DIM = 768
N_ART = 32 * 512
N_Q = 32 * 64
N_OPT = 32 * 5 * 48

NUM_CORES = 2
NUM_SUBCORES = 16
NW = NUM_CORES * NUM_SUBCORES


def _sc_gather(word_emb, ids, row_off, rows_per_w, chunk):
    n_rows = rows_per_w * NW
    n = rows_per_w // chunk
    mesh = plsc.VectorSubcoreMesh(core_axis_name="c", subcore_axis_name="s")

    @functools.partial(
        pl.kernel,
        mesh=mesh,
        out_type=jax.ShapeDtypeStruct((n_rows, DIM), jnp.float32),
        scratch_types=[
            pltpu.VMEM((rows_per_w,), jnp.int32),
            pltpu.VMEM((chunk, DIM), jnp.float32),
            pltpu.VMEM((chunk, DIM), jnp.float32),
            pltpu.SemaphoreType.DMA,
            pltpu.SemaphoreType.DMA,
            pltpu.SemaphoreType.DMA,
            pltpu.SemaphoreType.DMA,
        ],
    )
    def k(table_hbm, idx_hbm, out_hbm, idx_v, buf0, buf1, g0, g1, w0, w1):
        wid = lax.axis_index("s") * NUM_CORES + lax.axis_index("c")
        base = wid * rows_per_w
        pltpu.sync_copy(idx_hbm.at[pl.ds(row_off + base, rows_per_w)], idx_v)

        bufs = (buf0, buf1)
        gsems = (g0, g1)
        wsems = (w0, w1)

        def gather(c):
            cp = pltpu.make_async_copy(
                table_hbm.at[idx_v.at[pl.ds(c * chunk, chunk)]],
                bufs[c % 2], gsems[c % 2])
            cp.start()
            return cp

        def write(c):
            cp = pltpu.make_async_copy(
                bufs[c % 2],
                out_hbm.at[pl.ds(base + c * chunk, chunk)],
                wsems[c % 2])
            cp.start()
            return cp

        gathers = [gather(0)]
        writes = []
        for c in range(n):
            if c + 1 < n:
                if c >= 1:
                    writes[c - 1].wait()
                gathers.append(gather(c + 1))
            gathers[c].wait()
            writes.append(write(c))
        writes[n - 1].wait()
        if n > 1:
            writes[n - 2].wait()

    return k(word_emb, ids)


def _ln_body(g_ref, add_ref, gam_ref, bet_ref, o_ref):
    rows = g_ref.shape[0]
    add = add_ref[...]
    if add.shape[0] != rows:
        reps = rows // add.shape[0]
        add = jnp.broadcast_to(add[None], (reps,) + add.shape).reshape(rows,
                                                                       DIM)
    x = g_ref[...] + add
    mu = jnp.mean(x, axis=1, keepdims=True)
    xc = x - mu
    var = jnp.mean(xc * xc, axis=1, keepdims=True)
    o_ref[...] = xc * lax.rsqrt(var + 1e-12) * gam_ref[...] + bet_ref[...]


def _ln_alias_body(g_ref, _old_ref, add_ref, gam_ref, bet_ref, o_ref):
    _ln_body(g_ref, add_ref, gam_ref, bet_ref, o_ref)


def _ln_call(gathered, addtab, gamma2d, beta2d, block, out_rows=None):
    nrows = gathered.shape[0]
    grid = nrows // block
    add_rows = addtab.shape[0]
    if out_rows is None:
        out_rows = nrows
    return pl.pallas_call(
        _ln_body,
        grid=(grid,),
        in_specs=[
            pl.BlockSpec((block, DIM), lambda i: (i, 0)),
            pl.BlockSpec((add_rows, DIM), lambda i: (0, 0)),
            pl.BlockSpec((1, DIM), lambda i: (0, 0)),
            pl.BlockSpec((1, DIM), lambda i: (0, 0)),
        ],
        out_specs=pl.BlockSpec((block, DIM), lambda i: (i, 0)),
        out_shape=jax.ShapeDtypeStruct((out_rows, DIM), jnp.float32),
    )(gathered, addtab, gamma2d, beta2d)


def _ln_call_alias(gathered, partial_out, addtab, gamma2d, beta2d, block,
                   blk_off):
    nrows = gathered.shape[0]
    grid = nrows // block
    add_rows = addtab.shape[0]
    return pl.pallas_call(
        _ln_alias_body,
        grid=(grid,),
        in_specs=[
            pl.BlockSpec((block, DIM), lambda i: (i, 0)),
            pl.BlockSpec((block, DIM), lambda i: (i + blk_off, 0)),
            pl.BlockSpec((add_rows, DIM), lambda i: (0, 0)),
            pl.BlockSpec((1, DIM), lambda i: (0, 0)),
            pl.BlockSpec((1, DIM), lambda i: (0, 0)),
        ],
        out_specs=pl.BlockSpec((block, DIM), lambda i: (i + blk_off, 0)),
        out_shape=jax.ShapeDtypeStruct(partial_out.shape, jnp.float32),
        input_output_aliases={1: 0},
    )(gathered, partial_out, addtab, gamma2d, beta2d)


def kernel(article_tokens, question_tokens, options_tokens, word_emb,
           pos_emb, tok_type_emb, gamma, beta):
    art_ids = article_tokens.reshape(-1).astype(jnp.int32)
    q_ids = question_tokens.reshape(-1).astype(jnp.int32)
    opt_ids = options_tokens.reshape(-1).astype(jnp.int32)
    half = N_ART // 2

    ga1 = _sc_gather(word_emb, art_ids, 0, rows_per_w=256, chunk=64)
    go = _sc_gather(word_emb, opt_ids, 0, rows_per_w=240, chunk=40)
    ga2 = _sc_gather(word_emb, art_ids, half, rows_per_w=256, chunk=64)
    gq = _sc_gather(word_emb, q_ids, 0, rows_per_w=64, chunk=64)

    addvec = pos_emb + tok_type_emb[0]
    q_add = addvec[:64]
    o_add = addvec[:1]
    g2 = gamma.reshape(1, DIM)
    b2 = beta.reshape(1, DIM)

    art1 = _ln_call(ga1, addvec, g2, b2, block=512, out_rows=N_ART)
    opt = _ln_call(go, o_add, g2, b2, block=512)
    art = _ln_call_alias(ga2, art1, addvec, g2, b2, block=512,
                         blk_off=half // 512)
    q = _ln_call(gq, q_add, g2, b2, block=512)

    return (art.reshape(32, 512, DIM),
            q.reshape(32, 64, DIM),
            opt.reshape(32, 5, 48, DIM))


# speedup vs baseline: 1.1862x; 1.0482x over previous
import functools

import jax
import jax.numpy as jnp
from jax import lax
from jax.experimental import pallas as pl
from jax.experimental.pallas import tpu as pltpu
from jax.experimental.pallas import tpu_sc as plsc

DIM = 768
N_ART = 32 * 512
N_Q = 32 * 64
N_OPT = 32 * 5 * 48

NUM_CORES = 2
NUM_SUBCORES = 16
NW = NUM_CORES * NUM_SUBCORES


def _sc_gather(word_emb, ids, row_off, rows_per_w, chunk):
    n_rows = rows_per_w * NW
    n = rows_per_w // chunk
    mesh = plsc.VectorSubcoreMesh(core_axis_name="c", subcore_axis_name="s")

    @functools.partial(
        pl.kernel,
        mesh=mesh,
        out_type=jax.ShapeDtypeStruct((n_rows, DIM), jnp.float32),
        scratch_types=[
            pltpu.VMEM((rows_per_w,), jnp.int32),
            pltpu.VMEM((chunk, DIM), jnp.float32),
            pltpu.VMEM((chunk, DIM), jnp.float32),
            pltpu.SemaphoreType.DMA,
            pltpu.SemaphoreType.DMA,
            pltpu.SemaphoreType.DMA,
            pltpu.SemaphoreType.DMA,
        ],
    )
    def k(table_hbm, idx_hbm, out_hbm, idx_v, buf0, buf1, g0, g1, w0, w1):
        wid = lax.axis_index("s") * NUM_CORES + lax.axis_index("c")
        base = wid * rows_per_w
        pltpu.sync_copy(idx_hbm.at[pl.ds(row_off + base, rows_per_w)], idx_v)

        bufs = (buf0, buf1)
        gsems = (g0, g1)
        wsems = (w0, w1)

        def gather(c):
            cp = pltpu.make_async_copy(
                table_hbm.at[idx_v.at[pl.ds(c * chunk, chunk)]],
                bufs[c % 2], gsems[c % 2])
            cp.start()
            return cp

        def write(c):
            cp = pltpu.make_async_copy(
                bufs[c % 2],
                out_hbm.at[pl.ds(base + c * chunk, chunk)],
                wsems[c % 2])
            cp.start()
            return cp

        gathers = [gather(0)]
        writes = []
        for c in range(n):
            if c + 1 < n:
                if c >= 1:
                    writes[c - 1].wait()
                gathers.append(gather(c + 1))
            gathers[c].wait()
            writes.append(write(c))
        writes[n - 1].wait()
        if n > 1:
            writes[n - 2].wait()

    return k(word_emb, ids)


def _ln_body(g_ref, add_ref, gam_ref, bet_ref, o_ref):
    rows = g_ref.shape[0]
    add = add_ref[...]
    if add.shape[0] != rows:
        reps = rows // add.shape[0]
        add = jnp.broadcast_to(add[None], (reps,) + add.shape).reshape(rows,
                                                                       DIM)
    x = g_ref[...] + add
    mu = jnp.mean(x, axis=1, keepdims=True)
    xc = x - mu
    var = jnp.mean(xc * xc, axis=1, keepdims=True)
    o_ref[...] = xc * lax.rsqrt(var + 1e-12) * gam_ref[...] + bet_ref[...]


def _ln_alias_body(g_ref, _old_ref, add_ref, gam_ref, bet_ref, o_ref):
    _ln_body(g_ref, add_ref, gam_ref, bet_ref, o_ref)


def _ln_call(gathered, addtab, gamma2d, beta2d, block, out_rows=None):
    nrows = gathered.shape[0]
    grid = nrows // block
    add_rows = addtab.shape[0]
    if out_rows is None:
        out_rows = nrows
    return pl.pallas_call(
        _ln_body,
        grid=(grid,),
        in_specs=[
            pl.BlockSpec((block, DIM), lambda i: (i, 0)),
            pl.BlockSpec((add_rows, DIM), lambda i: (0, 0)),
            pl.BlockSpec((1, DIM), lambda i: (0, 0)),
            pl.BlockSpec((1, DIM), lambda i: (0, 0)),
        ],
        out_specs=pl.BlockSpec((block, DIM), lambda i: (i, 0)),
        out_shape=jax.ShapeDtypeStruct((out_rows, DIM), jnp.float32),
    )(gathered, addtab, gamma2d, beta2d)


def _ln_call_alias(gathered, partial_out, addtab, gamma2d, beta2d, block,
                   blk_off):
    nrows = gathered.shape[0]
    grid = nrows // block
    add_rows = addtab.shape[0]
    return pl.pallas_call(
        _ln_alias_body,
        grid=(grid,),
        in_specs=[
            pl.BlockSpec((block, DIM), lambda i: (i, 0)),
            pl.BlockSpec((8, 128), lambda i: (0, 0)),
            pl.BlockSpec((add_rows, DIM), lambda i: (0, 0)),
            pl.BlockSpec((1, DIM), lambda i: (0, 0)),
            pl.BlockSpec((1, DIM), lambda i: (0, 0)),
        ],
        out_specs=pl.BlockSpec((block, DIM), lambda i: (i + blk_off, 0)),
        out_shape=jax.ShapeDtypeStruct(partial_out.shape, jnp.float32),
        input_output_aliases={1: 0},
    )(gathered, partial_out, addtab, gamma2d, beta2d)


def kernel(article_tokens, question_tokens, options_tokens, word_emb,
           pos_emb, tok_type_emb, gamma, beta):
    art_ids = article_tokens.reshape(-1).astype(jnp.int32)
    q_ids = question_tokens.reshape(-1).astype(jnp.int32)
    opt_ids = options_tokens.reshape(-1).astype(jnp.int32)
    half = N_ART // 2

    ga1 = _sc_gather(word_emb, art_ids, 0, rows_per_w=256, chunk=64)
    go = _sc_gather(word_emb, opt_ids, 0, rows_per_w=240, chunk=40)
    ga2 = _sc_gather(word_emb, art_ids, half, rows_per_w=256, chunk=64)
    gq = _sc_gather(word_emb, q_ids, 0, rows_per_w=64, chunk=64)

    addvec = pos_emb + tok_type_emb[0]
    q_add = addvec[:64]
    o_add = addvec[:1]
    g2 = gamma.reshape(1, DIM)
    b2 = beta.reshape(1, DIM)

    art1 = _ln_call(ga1, addvec, g2, b2, block=512, out_rows=N_ART)
    opt = _ln_call(go, o_add, g2, b2, block=512)
    art = _ln_call_alias(ga2, art1, addvec, g2, b2, block=512,
                         blk_off=half // 512)
    q = _ln_call(gq, q_add, g2, b2, block=512)

    return (art.reshape(32, 512, DIM),
            q.reshape(32, 64, DIM),
            opt.reshape(32, 5, 48, DIM))


# speedup vs baseline: 1.1875x; 1.0011x over previous
import functools

import jax
import jax.numpy as jnp
from jax import lax
from jax.experimental import pallas as pl
from jax.experimental.pallas import tpu as pltpu
from jax.experimental.pallas import tpu_sc as plsc

DIM = 768
N_ART = 32 * 512
N_Q = 32 * 64
N_OPT = 32 * 5 * 48

NUM_CORES = 2
NUM_SUBCORES = 16
NW = NUM_CORES * NUM_SUBCORES


def _sc_gather(word_emb, ids, row_off, rows_per_w, chunk):
    n_rows = rows_per_w * NW
    n = rows_per_w // chunk
    mesh = plsc.VectorSubcoreMesh(core_axis_name="c", subcore_axis_name="s")

    @functools.partial(
        pl.kernel,
        mesh=mesh,
        out_type=jax.ShapeDtypeStruct((n_rows, DIM), jnp.float32),
        scratch_types=[
            pltpu.VMEM((rows_per_w,), jnp.int32),
            pltpu.VMEM((chunk, DIM), jnp.float32),
            pltpu.VMEM((chunk, DIM), jnp.float32),
            pltpu.SemaphoreType.DMA,
            pltpu.SemaphoreType.DMA,
            pltpu.SemaphoreType.DMA,
            pltpu.SemaphoreType.DMA,
        ],
    )
    def k(table_hbm, idx_hbm, out_hbm, idx_v, buf0, buf1, g0, g1, w0, w1):
        wid = lax.axis_index("s") * NUM_CORES + lax.axis_index("c")
        base = wid * rows_per_w
        pltpu.sync_copy(idx_hbm.at[pl.ds(row_off + base, rows_per_w)], idx_v)

        bufs = (buf0, buf1)
        gsems = (g0, g1)
        wsems = (w0, w1)

        def gather(c):
            cp = pltpu.make_async_copy(
                table_hbm.at[idx_v.at[pl.ds(c * chunk, chunk)]],
                bufs[c % 2], gsems[c % 2])
            cp.start()
            return cp

        def write(c):
            cp = pltpu.make_async_copy(
                bufs[c % 2],
                out_hbm.at[pl.ds(base + c * chunk, chunk)],
                wsems[c % 2])
            cp.start()
            return cp

        gathers = [gather(0)]
        writes = []
        for c in range(n):
            if c + 1 < n:
                if c >= 1:
                    writes[c - 1].wait()
                gathers.append(gather(c + 1))
            gathers[c].wait()
            writes.append(write(c))
        writes[n - 1].wait()
        if n > 1:
            writes[n - 2].wait()

    return k(word_emb, ids)


def _ln_body(g_ref, add_ref, gam_ref, bet_ref, o_ref):
    rows = g_ref.shape[0]
    add = add_ref[...]
    if add.shape[0] != rows:
        reps = rows // add.shape[0]
        add = jnp.broadcast_to(add[None], (reps,) + add.shape).reshape(rows,
                                                                       DIM)
    x = g_ref[...] + add
    mu = jnp.mean(x, axis=1, keepdims=True)
    xc = x - mu
    var = jnp.mean(xc * xc, axis=1, keepdims=True)
    o_ref[...] = xc * lax.rsqrt(var + 1e-12) * gam_ref[...] + bet_ref[...]


def _ln_alias_body(g_ref, _old_ref, add_ref, gam_ref, bet_ref, o_ref):
    _ln_body(g_ref, add_ref, gam_ref, bet_ref, o_ref)


def _ln_call(gathered, addtab, gamma2d, beta2d, block, out_rows=None):
    nrows = gathered.shape[0]
    grid = nrows // block
    add_rows = addtab.shape[0]
    if out_rows is None:
        out_rows = nrows
    return pl.pallas_call(
        _ln_body,
        grid=(grid,),
        in_specs=[
            pl.BlockSpec((block, DIM), lambda i: (i, 0)),
            pl.BlockSpec((add_rows, DIM), lambda i: (0, 0)),
            pl.BlockSpec((1, DIM), lambda i: (0, 0)),
            pl.BlockSpec((1, DIM), lambda i: (0, 0)),
        ],
        out_specs=pl.BlockSpec((block, DIM), lambda i: (i, 0)),
        out_shape=jax.ShapeDtypeStruct((out_rows, DIM), jnp.float32),
    )(gathered, addtab, gamma2d, beta2d)


def _ln_call_alias(gathered, partial_out, addtab, gamma2d, beta2d, block,
                   blk_off):
    nrows = gathered.shape[0]
    grid = nrows // block
    add_rows = addtab.shape[0]
    return pl.pallas_call(
        _ln_alias_body,
        grid=(grid,),
        in_specs=[
            pl.BlockSpec((block, DIM), lambda i: (i, 0)),
            pl.BlockSpec((8, 128), lambda i: (0, 0)),
            pl.BlockSpec((add_rows, DIM), lambda i: (0, 0)),
            pl.BlockSpec((1, DIM), lambda i: (0, 0)),
            pl.BlockSpec((1, DIM), lambda i: (0, 0)),
        ],
        out_specs=pl.BlockSpec((block, DIM), lambda i: (i + blk_off, 0)),
        out_shape=jax.ShapeDtypeStruct(partial_out.shape, jnp.float32),
        input_output_aliases={1: 0},
    )(gathered, partial_out, addtab, gamma2d, beta2d)


def kernel(article_tokens, question_tokens, options_tokens, word_emb,
           pos_emb, tok_type_emb, gamma, beta):
    art_ids = article_tokens.reshape(-1).astype(jnp.int32)
    q_ids = question_tokens.reshape(-1).astype(jnp.int32)
    opt_ids = options_tokens.reshape(-1).astype(jnp.int32)
    half = N_ART // 2

    ga1 = _sc_gather(word_emb, art_ids, 0, rows_per_w=256, chunk=64)
    goh = _sc_gather(word_emb, opt_ids, 0, rows_per_w=80, chunk=40)
    gor = _sc_gather(word_emb, opt_ids, 2560, rows_per_w=160, chunk=40)
    ga2 = _sc_gather(word_emb, art_ids, half, rows_per_w=256, chunk=64)
    gq = _sc_gather(word_emb, q_ids, 0, rows_per_w=64, chunk=64)

    addvec = pos_emb + tok_type_emb[0]
    q_add = addvec[:64]
    o_add = addvec[:1]
    g2 = gamma.reshape(1, DIM)
    b2 = beta.reshape(1, DIM)

    art1 = _ln_call(ga1, addvec, g2, b2, block=512, out_rows=N_ART)
    opt1 = _ln_call(goh, o_add, g2, b2, block=512, out_rows=N_OPT)
    opt = _ln_call_alias(gor, opt1, o_add, g2, b2, block=512, blk_off=5)
    art = _ln_call_alias(ga2, art1, addvec, g2, b2, block=512,
                         blk_off=half // 512)
    q = _ln_call(gq, q_add, g2, b2, block=512)

    return (art.reshape(32, 512, DIM),
            q.reshape(32, 64, DIM),
            opt.reshape(32, 5, 48, DIM))


# speedup vs baseline: 1.1878x; 1.0002x over previous
import functools

import jax
import jax.numpy as jnp
from jax import lax
from jax.experimental import pallas as pl
from jax.experimental.pallas import tpu as pltpu
from jax.experimental.pallas import tpu_sc as plsc

DIM = 768
N_ART = 32 * 512
N_Q = 32 * 64
N_OPT = 32 * 5 * 48

NUM_CORES = 2
NUM_SUBCORES = 16
NW = NUM_CORES * NUM_SUBCORES


def _sc_gather(word_emb, ids, row_off, rows_per_w, chunk):
    n_rows = rows_per_w * NW
    n = rows_per_w // chunk
    mesh = plsc.VectorSubcoreMesh(core_axis_name="c", subcore_axis_name="s")

    @functools.partial(
        pl.kernel,
        mesh=mesh,
        out_type=jax.ShapeDtypeStruct((n_rows, DIM), jnp.float32),
        scratch_types=[
            pltpu.VMEM((rows_per_w,), jnp.int32),
            pltpu.VMEM((chunk, DIM), jnp.float32),
            pltpu.VMEM((chunk, DIM), jnp.float32),
            pltpu.SemaphoreType.DMA,
            pltpu.SemaphoreType.DMA,
            pltpu.SemaphoreType.DMA,
            pltpu.SemaphoreType.DMA,
        ],
    )
    def k(table_hbm, idx_hbm, out_hbm, idx_v, buf0, buf1, g0, g1, w0, w1):
        wid = lax.axis_index("s") * NUM_CORES + lax.axis_index("c")
        base = wid * rows_per_w
        pltpu.sync_copy(idx_hbm.at[pl.ds(row_off + base, rows_per_w)], idx_v)

        bufs = (buf0, buf1)
        gsems = (g0, g1)
        wsems = (w0, w1)

        def gather(c):
            cp = pltpu.make_async_copy(
                table_hbm.at[idx_v.at[pl.ds(c * chunk, chunk)]],
                bufs[c % 2], gsems[c % 2])
            cp.start()
            return cp

        def write(c):
            cp = pltpu.make_async_copy(
                bufs[c % 2],
                out_hbm.at[pl.ds(base + c * chunk, chunk)],
                wsems[c % 2])
            cp.start()
            return cp

        gathers = [gather(0)]
        writes = []
        for c in range(n):
            if c + 1 < n:
                if c >= 1:
                    writes[c - 1].wait()
                gathers.append(gather(c + 1))
            gathers[c].wait()
            writes.append(write(c))
        writes[n - 1].wait()
        if n > 1:
            writes[n - 2].wait()

    return k(word_emb, ids)


def _ln_body(g_ref, add_ref, gam_ref, bet_ref, o_ref):
    rows = g_ref.shape[0]
    add = add_ref[...]
    if add.shape[0] != rows:
        reps = rows // add.shape[0]
        add = jnp.broadcast_to(add[None], (reps,) + add.shape).reshape(rows,
                                                                       DIM)
    x = g_ref[...] + add
    mu = jnp.mean(x, axis=1, keepdims=True)
    xc = x - mu
    var = jnp.mean(xc * xc, axis=1, keepdims=True)
    o_ref[...] = xc * lax.rsqrt(var + 1e-12) * gam_ref[...] + bet_ref[...]


def _ln_alias_body(g_ref, _old_ref, add_ref, gam_ref, bet_ref, o_ref):
    _ln_body(g_ref, add_ref, gam_ref, bet_ref, o_ref)


def _ln_call(gathered, addtab, gamma2d, beta2d, block, out_rows=None):
    nrows = gathered.shape[0]
    grid = nrows // block
    add_rows = addtab.shape[0]
    if out_rows is None:
        out_rows = nrows
    return pl.pallas_call(
        _ln_body,
        grid=(grid,),
        in_specs=[
            pl.BlockSpec((block, DIM), lambda i: (i, 0)),
            pl.BlockSpec((add_rows, DIM), lambda i: (0, 0)),
            pl.BlockSpec((1, DIM), lambda i: (0, 0)),
            pl.BlockSpec((1, DIM), lambda i: (0, 0)),
        ],
        out_specs=pl.BlockSpec((block, DIM), lambda i: (i, 0)),
        out_shape=jax.ShapeDtypeStruct((out_rows, DIM), jnp.float32),
    )(gathered, addtab, gamma2d, beta2d)


def _ln_call_alias(gathered, partial_out, addtab, gamma2d, beta2d, block,
                   blk_off):
    nrows = gathered.shape[0]
    grid = nrows // block
    add_rows = addtab.shape[0]
    return pl.pallas_call(
        _ln_alias_body,
        grid=(grid,),
        in_specs=[
            pl.BlockSpec((block, DIM), lambda i: (i, 0)),
            pl.BlockSpec((8, 128), lambda i: (0, 0)),
            pl.BlockSpec((add_rows, DIM), lambda i: (0, 0)),
            pl.BlockSpec((1, DIM), lambda i: (0, 0)),
            pl.BlockSpec((1, DIM), lambda i: (0, 0)),
        ],
        out_specs=pl.BlockSpec((block, DIM), lambda i: (i + blk_off, 0)),
        out_shape=jax.ShapeDtypeStruct(partial_out.shape, jnp.float32),
        input_output_aliases={1: 0},
    )(gathered, partial_out, addtab, gamma2d, beta2d)


def kernel(article_tokens, question_tokens, options_tokens, word_emb,
           pos_emb, tok_type_emb, gamma, beta):
    art_ids = article_tokens.reshape(-1).astype(jnp.int32)
    q_ids = question_tokens.reshape(-1).astype(jnp.int32)
    opt_ids = options_tokens.reshape(-1).astype(jnp.int32)
    half = N_ART // 2

    goh = _sc_gather(word_emb, opt_ids, 0, rows_per_w=80, chunk=40)
    ga1 = _sc_gather(word_emb, art_ids, 0, rows_per_w=256, chunk=64)
    gor = _sc_gather(word_emb, opt_ids, 2560, rows_per_w=160, chunk=40)
    ga2 = _sc_gather(word_emb, art_ids, half, rows_per_w=256, chunk=64)
    gq = _sc_gather(word_emb, q_ids, 0, rows_per_w=64, chunk=64)

    addvec = pos_emb + tok_type_emb[0]
    q_add = addvec[:64]
    o_add = addvec[:1]
    g2 = gamma.reshape(1, DIM)
    b2 = beta.reshape(1, DIM)

    art1 = _ln_call(ga1, addvec, g2, b2, block=512, out_rows=N_ART)
    opt1 = _ln_call(goh, o_add, g2, b2, block=512, out_rows=N_OPT)
    opt = _ln_call_alias(gor, opt1, o_add, g2, b2, block=512, blk_off=5)
    art = _ln_call_alias(ga2, art1, addvec, g2, b2, block=512,
                         blk_off=half // 512)
    q = _ln_call(gq, q_add, g2, b2, block=512)

    return (art.reshape(32, 512, DIM),
            q.reshape(32, 64, DIM),
            opt.reshape(32, 5, 48, DIM))
